# baseline (speedup 1.0000x reference)
import functools

import jax
import jax.numpy as jnp
from jax import lax
from jax.experimental import pallas as pl
from jax.experimental.pallas import tpu as pltpu
from jax.experimental.pallas import tpu_sc as plsc

N = 10000
D = 128
E = 320000
C = 16

NTILES = 32
CHUNK = 64
NBUF = 4
CH_GROUP = 8
K0_CH = 312
K1_CH = 8
TOTAL_CH = 16 * (K0_CH + K1_CH)
EP = TOTAL_CH * CHUNK
NPAD = 10240
ROWS_PER_TILE = NPAD // 16
ZBLK = 128

_mesh = plsc.VectorSubcoreMesh(core_axis_name="c", subcore_axis_name="s")


@functools.partial(
    pl.kernel,
    mesh=_mesh,
    out_type=jax.ShapeDtypeStruct((2, NPAD, D), jnp.float32),
    scratch_types=[
        pltpu.VMEM((2, CH_GROUP, CHUNK), jnp.int32),
        pltpu.VMEM((2, CH_GROUP, CHUNK), jnp.int32),
        pltpu.VMEM((NBUF * CHUNK, D), jnp.float32),
        pltpu.VMEM_SHARED((NPAD, D), jnp.float32),
        pltpu.SemaphoreType.DMA,
        pltpu.SemaphoreType.DMA,
        pltpu.SemaphoreType.DMA,
        pltpu.SemaphoreType.DMA,
        pltpu.SemaphoreType.DMA,
        pltpu.SemaphoreType.DMA,
    ],
)
def _sc_agg(h_hbm, src_hbm, dst_hbm, out_hbm,
            srcw, dstw, buf, acc, semg0, semg1, semg2, semg3, sems, semw):
    cid = lax.axis_index("c")
    sid = lax.axis_index("s")
    semg = [semg0, semg1, semg2, semg3]

    def _bufsl(b):
        return buf.at[pl.ds(b * CHUNK, CHUNK)]

    def _zrow(i, _):
        for k in range(D // 16):
            buf[i, pl.ds(k * 16, 16)] = jnp.zeros((16,), jnp.float32)
        return 0
    row0 = sid * ROWS_PER_TILE
    lax.fori_loop(0, ZBLK, _zrow, 0)
    for r in range(ROWS_PER_TILE // ZBLK):
        pltpu.sync_copy(buf.at[pl.ds(0, ZBLK)],
                        acc.at[pl.ds(row0 + r * ZBLK, ZBLK)])
    plsc.subcore_barrier()

    k_ch = jnp.where(cid == 0, K0_CH, K1_CH)
    n_grp = k_ch // CH_GROUP
    base_ch = jnp.where(cid == 0, sid * K0_CH, 16 * K0_CH + sid * K1_CH)

    @pl.when(k_ch > 0)
    def _():
        pltpu.sync_copy(src_hbm.at[pl.ds(base_ch, CH_GROUP)], srcw.at[0])
        pltpu.sync_copy(dst_hbm.at[pl.ds(base_ch, CH_GROUP)], dstw.at[0])

    def _group(g, _):
        p = g % 2
        c1 = base_ch + (g + 1) * CH_GROUP

        @pl.when(g + 1 < n_grp)
        def _():
            pltpu.async_copy(src_hbm.at[pl.ds(c1, CH_GROUP)],
                             srcw.at[1 - p], semw)
            pltpu.async_copy(dst_hbm.at[pl.ds(c1, CH_GROUP)],
                             dstw.at[1 - p], semw)

        for q in range(CH_GROUP // NBUF):
            for b in range(NBUF):
                row = q * NBUF + b
                pltpu.async_copy(h_hbm.at[srcw.at[p, row]], _bufsl(b),
                                 semg[b])
            for b in range(NBUF):
                row = q * NBUF + b
                pltpu.make_async_copy(h_hbm.at[srcw.at[p, row]], _bufsl(b),
                                      semg[b]).wait()
                pltpu.async_copy(_bufsl(b), acc.at[dstw.at[p, row]], sems,
                                 add=True)
            for b in range(NBUF):
                row = q * NBUF + b
                pltpu.make_async_copy(_bufsl(b), acc.at[dstw.at[p, row]],
                                      sems).wait()

        @pl.when(g + 1 < n_grp)
        def _():
            pltpu.make_async_copy(src_hbm.at[pl.ds(c1, CH_GROUP)],
                                  srcw.at[1 - p], semw).wait()
            pltpu.make_async_copy(dst_hbm.at[pl.ds(c1, CH_GROUP)],
                                  dstw.at[1 - p], semw).wait()
        return 0

    lax.fori_loop(0, n_grp, _group, 0)
    plsc.subcore_barrier()

    for r in range(ROWS_PER_TILE // ZBLK):
        pltpu.sync_copy(acc.at[pl.ds(row0 + r * ZBLK, ZBLK)],
                        out_hbm.at[cid, pl.ds(row0 + r * ZBLK, ZBLK)])


_BLK = 1000
_GRID = N // _BLK


def _colsum_body(x_ref, xi_ref, gx_ref, gi_ref):
    i = pl.program_id(0)
    sx = jnp.sum(x_ref[...], axis=0, keepdims=True)
    si = jnp.sum(xi_ref[...], axis=0, keepdims=True)

    @pl.when(i == 0)
    def _():
        gx_ref[...] = sx
        gi_ref[...] = si

    @pl.when(i > 0)
    def _():
        gx_ref[...] += sx
        gi_ref[...] += si


_colsums = pl.pallas_call(
    _colsum_body,
    grid=(_GRID,),
    in_specs=[
        pl.BlockSpec((_BLK, D), lambda i: (i, 0)),
        pl.BlockSpec((_BLK, D), lambda i: (i, 0)),
    ],
    out_specs=[
        pl.BlockSpec((1, D), lambda i: (0, 0)),
        pl.BlockSpec((1, D), lambda i: (0, 0)),
    ],
    out_shape=[
        jax.ShapeDtypeStruct((1, D), jnp.float32),
        jax.ShapeDtypeStruct((1, D), jnp.float32),
    ],
)


def _layer_body(eps_ref, g_ref, gi_ref, h_ref, agg_ref, lead_ref,
                w1_ref, b1_ref, w2_ref, b2_ref, h_out_ref, gsum_ref):
    i = pl.program_id(0)
    eps = eps_ref[0]
    z = ((1.0 + eps) * h_ref[...] + agg_ref[0] + agg_ref[1]
         + lead_ref[...] + g_ref[...] + gi_ref[...])
    z = jnp.maximum(
        jnp.dot(z, w1_ref[...], preferred_element_type=jnp.float32)
        + b1_ref[...], 0.0)
    hn = jnp.maximum(
        jnp.dot(z, w2_ref[...], preferred_element_type=jnp.float32)
        + b2_ref[...], 0.0)
    h_out_ref[...] = hn
    cs = jnp.sum(hn, axis=0, keepdims=True)

    @pl.when(i == 0)
    def _():
        gsum_ref[...] = cs

    @pl.when(i > 0)
    def _():
        gsum_ref[...] += cs


_layer = pl.pallas_call(
    _layer_body,
    grid=(_GRID,),
    in_specs=[
        pl.BlockSpec(memory_space=pltpu.SMEM),
        pl.BlockSpec((1, D), lambda i: (0, 0)),
        pl.BlockSpec((1, D), lambda i: (0, 0)),
        pl.BlockSpec((_BLK, D), lambda i: (i, 0)),
        pl.BlockSpec((2, _BLK, D), lambda i: (0, i, 0)),
        pl.BlockSpec((_BLK, D), lambda i: (i, 0)),
        pl.BlockSpec((D, D), lambda i: (0, 0)),
        pl.BlockSpec((1, D), lambda i: (0, 0)),
        pl.BlockSpec((D, D), lambda i: (0, 0)),
        pl.BlockSpec((1, D), lambda i: (0, 0)),
    ],
    out_specs=[
        pl.BlockSpec((_BLK, D), lambda i: (i, 0)),
        pl.BlockSpec((1, D), lambda i: (0, 0)),
    ],
    out_shape=[
        jax.ShapeDtypeStruct((N, D), jnp.float32),
        jax.ShapeDtypeStruct((1, D), jnp.float32),
    ],
)


def _final_body(eps_ref, g_ref, gi_ref, h_ref, agg_ref, lead_ref,
                w1_ref, b1_ref, w2_ref, b2_ref,
                f1w_ref, f1b_ref, f2w_ref, f2b_ref,
                out_ref, gsum_ref):
    i = pl.program_id(0)
    eps = eps_ref[0]
    z = ((1.0 + eps) * h_ref[...] + agg_ref[0] + agg_ref[1]
         + lead_ref[...] + g_ref[...] + gi_ref[...])
    z = jnp.maximum(
        jnp.dot(z, w1_ref[...], preferred_element_type=jnp.float32)
        + b1_ref[...], 0.0)
    hn = jnp.maximum(
        jnp.dot(z, w2_ref[...], preferred_element_type=jnp.float32)
        + b2_ref[...], 0.0)
    cs = jnp.sum(hn, axis=0, keepdims=True)

    @pl.when(i == 0)
    def _():
        gsum_ref[...] = cs

    @pl.when(i > 0)
    def _():
        gsum_ref[...] += cs

    @pl.when(i == _GRID - 1)
    def _():
        g3 = gsum_ref[...]
        t = jnp.maximum(
            jnp.dot(g3, f1w_ref[...], preferred_element_type=jnp.float32)
            + f1b_ref[...], 0.0)
        out_ref[...] = (
            jnp.dot(t, f2w_ref[...], preferred_element_type=jnp.float32)
            + f2b_ref[...])


_final = pl.pallas_call(
    _final_body,
    grid=(_GRID,),
    in_specs=[
        pl.BlockSpec(memory_space=pltpu.SMEM),
        pl.BlockSpec((1, D), lambda i: (0, 0)),
        pl.BlockSpec((1, D), lambda i: (0, 0)),
        pl.BlockSpec((_BLK, D), lambda i: (i, 0)),
        pl.BlockSpec((2, _BLK, D), lambda i: (0, i, 0)),
        pl.BlockSpec((_BLK, D), lambda i: (i, 0)),
        pl.BlockSpec((D, D), lambda i: (0, 0)),
        pl.BlockSpec((1, D), lambda i: (0, 0)),
        pl.BlockSpec((D, D), lambda i: (0, 0)),
        pl.BlockSpec((1, D), lambda i: (0, 0)),
        pl.BlockSpec((D, D), lambda i: (0, 0)),
        pl.BlockSpec((1, D), lambda i: (0, 0)),
        pl.BlockSpec((D, C), lambda i: (0, 0)),
        pl.BlockSpec((1, C), lambda i: (0, 0)),
    ],
    out_specs=[
        pl.BlockSpec((1, C), lambda i: (0, 0)),
        pl.BlockSpec((1, D), lambda i: (0, 0)),
    ],
    out_shape=[
        jax.ShapeDtypeStruct((1, C), jnp.float32),
        jax.ShapeDtypeStruct((1, D), jnp.float32),
    ],
)


def _agg_partials(h, src3, dst3):
    return _sc_agg(h, src3, dst3)


def kernel(x, edge_index, x_initial, x_lead,
           eps0, l0_W1, l0_b1, l0_W2, l0_b2,
           eps1, l1_W1, l1_b1, l1_W2, l1_b2,
           eps2, l2_W1, l2_b1, l2_W2, l2_b2,
           fc1_W, fc1_b, fc2_W, fc2_b):
    pad = EP - E
    src3 = jnp.concatenate(
        [edge_index[0], jnp.zeros((pad,), jnp.int32)]).reshape(
            TOTAL_CH, CHUNK)
    dst3 = jnp.concatenate(
        [edge_index[1], jnp.full((pad,), N, jnp.int32)]).reshape(
            TOTAL_CH, CHUNK)

    g, g_init = _colsums(x, x_initial)

    layers = [
        (eps0, l0_W1, l0_b1, l0_W2, l0_b2),
        (eps1, l1_W1, l1_b1, l1_W2, l1_b2),
        (eps2, l2_W1, l2_b1, l2_W2, l2_b2),
    ]
    h = x
    for li, (eps, W1, b1, W2, b2) in enumerate(layers):
        parts = _agg_partials(h, src3, dst3)
        eps1d = jnp.reshape(eps, (1,))
        if li < 2:
            h, g = _layer(eps1d, g, g_init, h, parts, x_lead,
                          W1, jnp.reshape(b1, (1, D)),
                          W2, jnp.reshape(b2, (1, D)))
        else:
            out, _ = _final(eps1d, g, g_init, h, parts, x_lead,
                            W1, jnp.reshape(b1, (1, D)),
                            W2, jnp.reshape(b2, (1, D)),
                            fc1_W, jnp.reshape(fc1_b, (1, D)),
                            fc2_W, jnp.reshape(fc2_b, (1, C)))
    return out


# speedup vs baseline: 1.0994x; 1.0994x over previous
import functools

import jax
import jax.numpy as jnp
from jax import lax
from jax.experimental import pallas as pl
from jax.experimental.pallas import tpu as pltpu
from jax.experimental.pallas import tpu_sc as plsc

N = 10000
D = 128
E = 320000
C = 16

NTILES = 32
CHUNK = 64
NBUF = 4
CH_GROUP = 8
K0_CH = 296
K1_CH = 24
TOTAL_CH = 16 * (K0_CH + K1_CH)
EP = TOTAL_CH * CHUNK
NPAD = 10240
ROWS_PER_TILE = NPAD // 16
ZBLK = 128

_mesh = plsc.VectorSubcoreMesh(core_axis_name="c", subcore_axis_name="s")


@functools.partial(
    pl.kernel,
    mesh=_mesh,
    out_type=jax.ShapeDtypeStruct((2, NPAD, D), jnp.float32),
    scratch_types=[
        pltpu.VMEM((2, CH_GROUP, CHUNK), jnp.int32),
        pltpu.VMEM((2, CH_GROUP, CHUNK), jnp.int32),
        pltpu.VMEM((NBUF * CHUNK, D), jnp.float32),
        pltpu.VMEM_SHARED((NPAD, D), jnp.float32),
        pltpu.SemaphoreType.DMA,
        pltpu.SemaphoreType.DMA,
        pltpu.SemaphoreType.DMA,
        pltpu.SemaphoreType.DMA,
        pltpu.SemaphoreType.DMA,
        pltpu.SemaphoreType.DMA,
    ],
)
def _sc_agg(h_hbm, src_hbm, dst_hbm, out_hbm,
            srcw, dstw, buf, acc, semg0, semg1, semg2, semg3, sems, semw):
    cid = lax.axis_index("c")
    sid = lax.axis_index("s")
    semg = [semg0, semg1, semg2, semg3]

    def _bufsl(b):
        return buf.at[pl.ds(b * CHUNK, CHUNK)]

    def _zrow(i, _):
        for k in range(D // 16):
            buf[i, pl.ds(k * 16, 16)] = jnp.zeros((16,), jnp.float32)
        return 0
    row0 = sid * ROWS_PER_TILE
    lax.fori_loop(0, ZBLK, _zrow, 0)
    for r in range(ROWS_PER_TILE // ZBLK):
        pltpu.async_copy(buf.at[pl.ds(0, ZBLK)],
                        acc.at[pl.ds(row0 + r * ZBLK, ZBLK)], semw)
    for r in range(ROWS_PER_TILE // ZBLK):
        pltpu.make_async_copy(buf.at[pl.ds(0, ZBLK)],
                              acc.at[pl.ds(row0 + r * ZBLK, ZBLK)],
                              semw).wait()
    plsc.subcore_barrier()

    k_ch = jnp.where(cid == 0, K0_CH, K1_CH)
    n_grp = k_ch // CH_GROUP
    base_ch = jnp.where(cid == 0, sid * K0_CH, 16 * K0_CH + sid * K1_CH)

    @pl.when(k_ch > 0)
    def _():
        pltpu.async_copy(src_hbm.at[pl.ds(base_ch, CH_GROUP)], srcw.at[0],
                         semw)
        pltpu.async_copy(dst_hbm.at[pl.ds(base_ch, CH_GROUP)], dstw.at[0],
                         semw)
        pltpu.make_async_copy(src_hbm.at[pl.ds(base_ch, CH_GROUP)],
                              srcw.at[0], semw).wait()
        pltpu.make_async_copy(dst_hbm.at[pl.ds(base_ch, CH_GROUP)],
                              dstw.at[0], semw).wait()

    def _group(g, _):
        p = g % 2
        c1 = base_ch + (g + 1) * CH_GROUP

        @pl.when(g + 1 < n_grp)
        def _():
            pltpu.async_copy(src_hbm.at[pl.ds(c1, CH_GROUP)],
                             srcw.at[1 - p], semw)
            pltpu.async_copy(dst_hbm.at[pl.ds(c1, CH_GROUP)],
                             dstw.at[1 - p], semw)

        for q in range(CH_GROUP // NBUF):
            for b in range(NBUF):
                row = q * NBUF + b
                pltpu.async_copy(h_hbm.at[srcw.at[p, row]], _bufsl(b),
                                 semg[b])
            for b in range(NBUF):
                row = q * NBUF + b
                pltpu.make_async_copy(h_hbm.at[srcw.at[p, row]], _bufsl(b),
                                      semg[b]).wait()
                pltpu.async_copy(_bufsl(b), acc.at[dstw.at[p, row]], sems,
                                 add=True)
            for b in range(NBUF):
                row = q * NBUF + b
                pltpu.make_async_copy(_bufsl(b), acc.at[dstw.at[p, row]],
                                      sems).wait()

        @pl.when(g + 1 < n_grp)
        def _():
            pltpu.make_async_copy(src_hbm.at[pl.ds(c1, CH_GROUP)],
                                  srcw.at[1 - p], semw).wait()
            pltpu.make_async_copy(dst_hbm.at[pl.ds(c1, CH_GROUP)],
                                  dstw.at[1 - p], semw).wait()
        return 0

    lax.fori_loop(0, n_grp, _group, 0)
    plsc.subcore_barrier()

    for r in range(ROWS_PER_TILE // ZBLK):
        pltpu.async_copy(acc.at[pl.ds(row0 + r * ZBLK, ZBLK)],
                        out_hbm.at[cid, pl.ds(row0 + r * ZBLK, ZBLK)], semw)
    for r in range(ROWS_PER_TILE // ZBLK):
        pltpu.make_async_copy(acc.at[pl.ds(row0 + r * ZBLK, ZBLK)],
                              out_hbm.at[cid, pl.ds(row0 + r * ZBLK, ZBLK)],
                              semw).wait()


_BLK = 1000
_GRID = N // _BLK


def _colsum_body(x_ref, xi_ref, gx_ref, gi_ref):
    i = pl.program_id(0)
    sx = jnp.sum(x_ref[...], axis=0, keepdims=True)
    si = jnp.sum(xi_ref[...], axis=0, keepdims=True)

    @pl.when(i == 0)
    def _():
        gx_ref[...] = sx
        gi_ref[...] = si

    @pl.when(i > 0)
    def _():
        gx_ref[...] += sx
        gi_ref[...] += si


_colsums = pl.pallas_call(
    _colsum_body,
    grid=(_GRID,),
    in_specs=[
        pl.BlockSpec((_BLK, D), lambda i: (i, 0)),
        pl.BlockSpec((_BLK, D), lambda i: (i, 0)),
    ],
    out_specs=[
        pl.BlockSpec((1, D), lambda i: (0, 0)),
        pl.BlockSpec((1, D), lambda i: (0, 0)),
    ],
    out_shape=[
        jax.ShapeDtypeStruct((1, D), jnp.float32),
        jax.ShapeDtypeStruct((1, D), jnp.float32),
    ],
)


def _layer_body(eps_ref, g_ref, gi_ref, h_ref, agg_ref, lead_ref,
                w1_ref, b1_ref, w2_ref, b2_ref, h_out_ref, gsum_ref):
    i = pl.program_id(0)
    eps = eps_ref[0]
    z = ((1.0 + eps) * h_ref[...] + agg_ref[0] + agg_ref[1]
         + lead_ref[...] + g_ref[...] + gi_ref[...])
    z = jnp.maximum(
        jnp.dot(z, w1_ref[...], preferred_element_type=jnp.float32)
        + b1_ref[...], 0.0)
    hn = jnp.maximum(
        jnp.dot(z, w2_ref[...], preferred_element_type=jnp.float32)
        + b2_ref[...], 0.0)
    h_out_ref[...] = hn
    cs = jnp.sum(hn, axis=0, keepdims=True)

    @pl.when(i == 0)
    def _():
        gsum_ref[...] = cs

    @pl.when(i > 0)
    def _():
        gsum_ref[...] += cs


_layer = pl.pallas_call(
    _layer_body,
    grid=(_GRID,),
    in_specs=[
        pl.BlockSpec(memory_space=pltpu.SMEM),
        pl.BlockSpec((1, D), lambda i: (0, 0)),
        pl.BlockSpec((1, D), lambda i: (0, 0)),
        pl.BlockSpec((_BLK, D), lambda i: (i, 0)),
        pl.BlockSpec((2, _BLK, D), lambda i: (0, i, 0)),
        pl.BlockSpec((_BLK, D), lambda i: (i, 0)),
        pl.BlockSpec((D, D), lambda i: (0, 0)),
        pl.BlockSpec((1, D), lambda i: (0, 0)),
        pl.BlockSpec((D, D), lambda i: (0, 0)),
        pl.BlockSpec((1, D), lambda i: (0, 0)),
    ],
    out_specs=[
        pl.BlockSpec((_BLK, D), lambda i: (i, 0)),
        pl.BlockSpec((1, D), lambda i: (0, 0)),
    ],
    out_shape=[
        jax.ShapeDtypeStruct((N, D), jnp.float32),
        jax.ShapeDtypeStruct((1, D), jnp.float32),
    ],
)


def _final_body(eps_ref, g_ref, gi_ref, h_ref, agg_ref, lead_ref,
                w1_ref, b1_ref, w2_ref, b2_ref,
                f1w_ref, f1b_ref, f2w_ref, f2b_ref,
                out_ref, gsum_ref):
    i = pl.program_id(0)
    eps = eps_ref[0]
    z = ((1.0 + eps) * h_ref[...] + agg_ref[0] + agg_ref[1]
         + lead_ref[...] + g_ref[...] + gi_ref[...])
    z = jnp.maximum(
        jnp.dot(z, w1_ref[...], preferred_element_type=jnp.float32)
        + b1_ref[...], 0.0)
    hn = jnp.maximum(
        jnp.dot(z, w2_ref[...], preferred_element_type=jnp.float32)
        + b2_ref[...], 0.0)
    cs = jnp.sum(hn, axis=0, keepdims=True)

    @pl.when(i == 0)
    def _():
        gsum_ref[...] = cs

    @pl.when(i > 0)
    def _():
        gsum_ref[...] += cs

    @pl.when(i == _GRID - 1)
    def _():
        g3 = gsum_ref[...]
        t = jnp.maximum(
            jnp.dot(g3, f1w_ref[...], preferred_element_type=jnp.float32)
            + f1b_ref[...], 0.0)
        out_ref[...] = (
            jnp.dot(t, f2w_ref[...], preferred_element_type=jnp.float32)
            + f2b_ref[...])


_final = pl.pallas_call(
    _final_body,
    grid=(_GRID,),
    in_specs=[
        pl.BlockSpec(memory_space=pltpu.SMEM),
        pl.BlockSpec((1, D), lambda i: (0, 0)),
        pl.BlockSpec((1, D), lambda i: (0, 0)),
        pl.BlockSpec((_BLK, D), lambda i: (i, 0)),
        pl.BlockSpec((2, _BLK, D), lambda i: (0, i, 0)),
        pl.BlockSpec((_BLK, D), lambda i: (i, 0)),
        pl.BlockSpec((D, D), lambda i: (0, 0)),
        pl.BlockSpec((1, D), lambda i: (0, 0)),
        pl.BlockSpec((D, D), lambda i: (0, 0)),
        pl.BlockSpec((1, D), lambda i: (0, 0)),
        pl.BlockSpec((D, D), lambda i: (0, 0)),
        pl.BlockSpec((1, D), lambda i: (0, 0)),
        pl.BlockSpec((D, C), lambda i: (0, 0)),
        pl.BlockSpec((1, C), lambda i: (0, 0)),
    ],
    out_specs=[
        pl.BlockSpec((1, C), lambda i: (0, 0)),
        pl.BlockSpec((1, D), lambda i: (0, 0)),
    ],
    out_shape=[
        jax.ShapeDtypeStruct((1, C), jnp.float32),
        jax.ShapeDtypeStruct((1, D), jnp.float32),
    ],
)


def _agg_partials(h, src3, dst3):
    return _sc_agg(h, src3, dst3)


def kernel(x, edge_index, x_initial, x_lead,
           eps0, l0_W1, l0_b1, l0_W2, l0_b2,
           eps1, l1_W1, l1_b1, l1_W2, l1_b2,
           eps2, l2_W1, l2_b1, l2_W2, l2_b2,
           fc1_W, fc1_b, fc2_W, fc2_b):
    pad = EP - E
    src3 = jnp.concatenate(
        [edge_index[0], jnp.zeros((pad,), jnp.int32)]).reshape(
            TOTAL_CH, CHUNK)
    dst3 = jnp.concatenate(
        [edge_index[1], jnp.full((pad,), N, jnp.int32)]).reshape(
            TOTAL_CH, CHUNK)

    g, g_init = _colsums(x, x_initial)

    layers = [
        (eps0, l0_W1, l0_b1, l0_W2, l0_b2),
        (eps1, l1_W1, l1_b1, l1_W2, l1_b2),
        (eps2, l2_W1, l2_b1, l2_W2, l2_b2),
    ]
    h = x
    for li, (eps, W1, b1, W2, b2) in enumerate(layers):
        parts = _agg_partials(h, src3, dst3)
        eps1d = jnp.reshape(eps, (1,))
        if li < 2:
            h, g = _layer(eps1d, g, g_init, h, parts, x_lead,
                          W1, jnp.reshape(b1, (1, D)),
                          W2, jnp.reshape(b2, (1, D)))
        else:
            out, _ = _final(eps1d, g, g_init, h, parts, x_lead,
                            W1, jnp.reshape(b1, (1, D)),
                            W2, jnp.reshape(b2, (1, D)),
                            fc1_W, jnp.reshape(fc1_b, (1, D)),
                            fc2_W, jnp.reshape(fc2_b, (1, C)))
    return out
